# Initial kernel scaffold; baseline (speedup 1.0000x reference)
import jax
import jax.numpy as jnp
from jax.experimental import pallas as pl


def kernel(x, edge_index, memory, gcn_weight, gcn_bias, w_ih, w_hh, b_ih, b_hh):
    raise NotImplementedError("write your pallas kernel here")


# speedup vs baseline: 12.9128x; 12.9128x over previous
import functools

import jax
import jax.numpy as jnp
from jax import lax
from jax.experimental import pallas as pl
from jax.experimental.pallas import tpu as pltpu
from jax.experimental.pallas import tpu_sc as plsc

N = 10000
E = 320000
IN_CH = 128
MEM = 128
OUT = 128

NC = 2
NS = 16
NW = NC * NS
EPW = E // NW
K = 80
CH = EPW // K
NPT = N // NS
ZB = 125
HL = 16

_mesh = plsc.VectorSubcoreMesh(core_axis_name="c", subcore_axis_name="s")

HIGHEST = jax.lax.Precision.HIGHEST


@functools.partial(
    pl.kernel,
    out_type=jax.ShapeDtypeStruct((NW, NPT, OUT), jnp.float32),
    mesh=_mesh,
    scratch_types=[
        pltpu.VMEM((K,), jnp.int32),
        pltpu.VMEM((K, OUT), jnp.float32),
        pltpu.VMEM((ZB, OUT), jnp.float32),
        pltpu.VMEM_SHARED((N, OUT), jnp.float32),
    ],
)
def _sc_histogram(col_hbm, out_hbm, cidx_v, ones_v, zero_v, acc_sh):
    cid = lax.axis_index("c")
    sid = lax.axis_index("s")
    wid = cid * NS + sid

    def fill_ones(i, _):
        for k in range(OUT // 16):
            ones_v[i, pl.ds(k * 16, 16)] = jnp.ones((16,), jnp.float32)
        return 0

    lax.fori_loop(0, K, fill_ones, 0)

    def fill_zero(i, _):
        for k in range(OUT // 16):
            zero_v[i, pl.ds(k * 16, 16)] = jnp.zeros((16,), jnp.float32)
        return 0

    lax.fori_loop(0, ZB, fill_zero, 0)
    for c in range(NPT // ZB):
        pltpu.sync_copy(zero_v, acc_sh.at[pl.ds(sid * NPT + c * ZB, ZB)])
    plsc.subcore_barrier()

    def chunk(j, _):
        pltpu.sync_copy(col_hbm.at[wid, j], cidx_v)
        pltpu.sync_copy(ones_v, acc_sh.at[cidx_v], add=True)
        return 0

    lax.fori_loop(0, CH, chunk, 0)
    plsc.subcore_barrier()

    pltpu.sync_copy(acc_sh.at[pl.ds(sid * NPT, NPT)], out_hbm.at[wid])


@functools.partial(
    pl.kernel,
    out_type=jax.ShapeDtypeStruct((NW, NPT, OUT), jnp.float32),
    mesh=_mesh,
    scratch_types=[
        pltpu.VMEM((K,), jnp.int32),
        pltpu.VMEM((K,), jnp.int32),
        pltpu.VMEM((K, OUT), jnp.float32),
        pltpu.VMEM((ZB, OUT), jnp.float32),
        pltpu.VMEM_SHARED((N, OUT), jnp.float32),
        pltpu.SemaphoreType.DMA,
    ],
)
def _sc_edge_sum(ys_hbm, row_hbm, col_hbm, out_hbm,
                 ridx_v, cidx_v, buf_a, zero_v, acc_sh, sem_a):
    cid = lax.axis_index("c")
    sid = lax.axis_index("s")
    wid = cid * NS + sid

    def fill_zero(i, _):
        for k in range(OUT // 16):
            zero_v[i, pl.ds(k * 16, 16)] = jnp.zeros((16,), jnp.float32)
        return 0

    lax.fori_loop(0, ZB, fill_zero, 0)
    for c in range(NPT // ZB):
        pltpu.sync_copy(zero_v, acc_sh.at[pl.ds(sid * NPT + c * ZB, ZB)])
    plsc.subcore_barrier()

    def chunk(j, _):
        pltpu.sync_copy(row_hbm.at[wid, j], ridx_v)
        pltpu.sync_copy(col_hbm.at[wid, j], cidx_v)
        pltpu.async_copy(ys_hbm.at[ridx_v], buf_a, sem_a).wait()
        pltpu.sync_copy(buf_a, acc_sh.at[cidx_v], add=True)
        return 0

    lax.fori_loop(0, CH, chunk, 0)

    plsc.subcore_barrier()
    pltpu.sync_copy(acc_sh.at[pl.ds(sid * NPT, NPT)], out_hbm.at[wid])


BLK = 1000


def _mm1_body(x_ref, mem_ref, wt1_ref, wt2_ref, whht_ref, bhh_ref,
              xw_ref, gh_ref):
    xw = jnp.dot(x_ref[...], wt1_ref[...],
                 preferred_element_type=jnp.float32, precision=HIGHEST)
    xw = xw + jnp.dot(mem_ref[...], wt2_ref[...],
                      preferred_element_type=jnp.float32, precision=HIGHEST)
    xw_ref[...] = xw
    gh_ref[...] = jnp.dot(mem_ref[...], whht_ref[...],
                          preferred_element_type=jnp.float32,
                          precision=HIGHEST) + bhh_ref[...]


def _tc_mm1(x, memory, wt1, wt2, whht, bhh):
    grid = (N // BLK,)
    return pl.pallas_call(
        _mm1_body,
        grid=grid,
        in_specs=[
            pl.BlockSpec((BLK, IN_CH), lambda i: (i, 0)),
            pl.BlockSpec((BLK, MEM), lambda i: (i, 0)),
            pl.BlockSpec((IN_CH, OUT), lambda i: (0, 0)),
            pl.BlockSpec((MEM, OUT), lambda i: (0, 0)),
            pl.BlockSpec((MEM, 3 * MEM), lambda i: (0, 0)),
            pl.BlockSpec((1, 3 * MEM), lambda i: (0, 0)),
        ],
        out_specs=[
            pl.BlockSpec((BLK, OUT), lambda i: (i, 0)),
            pl.BlockSpec((BLK, 3 * MEM), lambda i: (i, 0)),
        ],
        out_shape=[
            jax.ShapeDtypeStruct((N, OUT), jnp.float32),
            jax.ShapeDtypeStruct((N, 3 * MEM), jnp.float32),
        ],
    )(x, memory, wt1, wt2, whht, bhh)


def _scale_body(d0_ref, d1_ref, xw_ref, ys_ref, dis_ref):
    deg = d0_ref[:, 0:1] + d1_ref[:, 0:1] + 1.0
    dis = jax.lax.rsqrt(deg)
    ys_ref[...] = dis * xw_ref[...]
    dis_ref[...] = dis


def _tc_scale(deg0, deg1, xw):
    grid = (N // BLK,)
    return pl.pallas_call(
        _scale_body,
        grid=grid,
        in_specs=[
            pl.BlockSpec((BLK, OUT), lambda i: (i, 0)),
            pl.BlockSpec((BLK, OUT), lambda i: (i, 0)),
            pl.BlockSpec((BLK, OUT), lambda i: (i, 0)),
        ],
        out_specs=[
            pl.BlockSpec((BLK, OUT), lambda i: (i, 0)),
            pl.BlockSpec((BLK, 1), lambda i: (i, 0)),
        ],
        out_shape=[
            jax.ShapeDtypeStruct((N, OUT), jnp.float32),
            jax.ShapeDtypeStruct((N, 1), jnp.float32),
        ],
    )(deg0, deg1, xw)


def _final_body(a0_ref, a1_ref, ys_ref, dis_ref, mem_ref, gh_ref,
                wiht_ref, bih_ref, gb_ref, nm_ref, msg_ref):
    msg = dis_ref[...] * (a0_ref[...] + a1_ref[...] + ys_ref[...]) + gb_ref[...]
    msg_ref[...] = msg
    gi = jnp.dot(msg, wiht_ref[...],
                 preferred_element_type=jnp.float32,
                 precision=HIGHEST) + bih_ref[...]
    gh = gh_ref[...]
    r = jax.nn.sigmoid(gi[:, 0:MEM] + gh[:, 0:MEM])
    z = jax.nn.sigmoid(gi[:, MEM:2 * MEM] + gh[:, MEM:2 * MEM])
    ng = jnp.tanh(gi[:, 2 * MEM:] + r * gh[:, 2 * MEM:])
    nm_ref[...] = (1.0 - z) * ng + z * mem_ref[...]


def _tc_final(acc0, acc1, ys, dis, memory, gh, wiht, bih, gbias):
    grid = (N // BLK,)
    return pl.pallas_call(
        _final_body,
        grid=grid,
        in_specs=[
            pl.BlockSpec((BLK, OUT), lambda i: (i, 0)),
            pl.BlockSpec((BLK, OUT), lambda i: (i, 0)),
            pl.BlockSpec((BLK, OUT), lambda i: (i, 0)),
            pl.BlockSpec((BLK, 1), lambda i: (i, 0)),
            pl.BlockSpec((BLK, MEM), lambda i: (i, 0)),
            pl.BlockSpec((BLK, 3 * MEM), lambda i: (i, 0)),
            pl.BlockSpec((OUT, 3 * MEM), lambda i: (0, 0)),
            pl.BlockSpec((1, 3 * MEM), lambda i: (0, 0)),
            pl.BlockSpec((1, OUT), lambda i: (0, 0)),
        ],
        out_specs=[
            pl.BlockSpec((BLK, MEM), lambda i: (i, 0)),
            pl.BlockSpec((BLK, OUT), lambda i: (i, 0)),
        ],
        out_shape=[
            jax.ShapeDtypeStruct((N, MEM), jnp.float32),
            jax.ShapeDtypeStruct((N, OUT), jnp.float32),
        ],
    )(acc0, acc1, ys, dis, memory, gh, wiht, bih, gbias)


def kernel(x, edge_index, memory, gcn_weight, gcn_bias, w_ih, w_hh, b_ih, b_hh):
    row = edge_index[0].astype(jnp.int32).reshape(NW, CH, K)
    col = edge_index[1].astype(jnp.int32).reshape(NW, CH, K)

    hist = _sc_histogram(col)
    deg0 = hist[:NS].reshape(N, OUT)
    deg1 = hist[NS:].reshape(N, OUT)

    wt1 = gcn_weight[:, :IN_CH].T
    wt2 = gcn_weight[:, IN_CH:].T
    whht = w_hh.T
    wiht = w_ih.T
    xw, gh = _tc_mm1(x, memory, wt1, wt2, whht, b_hh.reshape(1, -1))

    ys, dis = _tc_scale(deg0, deg1, xw)

    acc = _sc_edge_sum(ys, row, col)
    acc0 = acc[:NS].reshape(N, OUT)
    acc1 = acc[NS:].reshape(N, OUT)

    new_memory, message = _tc_final(
        acc0, acc1, ys, dis, memory, gh, wiht,
        b_ih.reshape(1, -1), gcn_bias.reshape(1, -1))
    return (new_memory, message)


# speedup vs baseline: 21.9947x; 1.7033x over previous
import functools

import jax
import jax.numpy as jnp
from jax import lax
from jax.experimental import pallas as pl
from jax.experimental.pallas import tpu as pltpu
from jax.experimental.pallas import tpu_sc as plsc

N = 10000
E = 320000
IN_CH = 128
MEM = 128
OUT = 128

NC = 2
NS = 16
NW = NC * NS
EPW = E // NW
K = 80
CH = EPW // K
NPT = N // NS
ZB = 125
CHP = 128
WH = 4
HL = 16

_mesh = plsc.VectorSubcoreMesh(core_axis_name="c", subcore_axis_name="s")

HIGHEST = jax.lax.Precision.HIGHEST


@functools.partial(
    pl.kernel,
    out_type=jax.ShapeDtypeStruct((NW, NPT, OUT), jnp.float32),
    mesh=_mesh,
    scratch_types=[
        pltpu.VMEM((CHP, K), jnp.int32),
        pltpu.VMEM((K, OUT), jnp.float32),
        pltpu.VMEM((ZB, OUT), jnp.float32),
        pltpu.VMEM_SHARED((N, OUT), jnp.float32),
        pltpu.SemaphoreType.DMA,
    ],
)
def _sc_histogram(col_hbm, out_hbm, cidx_v, ones_v, zero_v, acc_sh, sem):
    cid = lax.axis_index("c")
    sid = lax.axis_index("s")
    wid = cid * NS + sid

    def fill_ones(i, _):
        for k in range(OUT // 16):
            ones_v[i, pl.ds(k * 16, 16)] = jnp.ones((16,), jnp.float32)
        return 0

    lax.fori_loop(0, K, fill_ones, 0)

    def fill_zero(i, _):
        for k in range(OUT // 16):
            zero_v[i, pl.ds(k * 16, 16)] = jnp.zeros((16,), jnp.float32)
        return 0

    lax.fori_loop(0, ZB, fill_zero, 0)
    for c in range(NPT // ZB):
        pltpu.sync_copy(zero_v, acc_sh.at[pl.ds(sid * NPT + c * ZB, ZB)])
    pltpu.sync_copy(col_hbm.at[pl.ds(wid * CHP, CHP)], cidx_v)
    plsc.subcore_barrier()

    for w in range(WH):
        pltpu.async_copy(ones_v, acc_sh.at[cidx_v.at[w]], sem, add=True)

    def chunk(j, _):
        pltpu.make_async_copy(ones_v, acc_sh.at[cidx_v.at[j - WH]], sem).wait()
        pltpu.async_copy(ones_v, acc_sh.at[cidx_v.at[j]], sem, add=True)
        return 0

    lax.fori_loop(WH, CH, chunk, 0)
    for w in range(WH):
        pltpu.make_async_copy(
            ones_v, acc_sh.at[cidx_v.at[CH - WH + w]], sem).wait()
    plsc.subcore_barrier()

    pltpu.sync_copy(acc_sh.at[pl.ds(sid * NPT, NPT)], out_hbm.at[wid])


@functools.partial(
    pl.kernel,
    out_type=jax.ShapeDtypeStruct((NW, NPT, OUT), jnp.float32),
    mesh=_mesh,
    scratch_types=[
        pltpu.VMEM((CHP, K), jnp.int32),
        pltpu.VMEM((K,), jnp.int32),
        pltpu.VMEM((K,), jnp.int32),
        pltpu.VMEM((K, OUT), jnp.float32),
        pltpu.VMEM((K, OUT), jnp.float32),
        pltpu.VMEM_SHARED((N, OUT), jnp.float32),
        pltpu.SemaphoreType.DMA,
        pltpu.SemaphoreType.DMA,
        pltpu.SemaphoreType.DMA,
        pltpu.SemaphoreType.DMA,
        pltpu.SemaphoreType.DMA,
        pltpu.SemaphoreType.DMA,
    ],
)
def _sc_edge_sum(ys_hbm, row_hbm, col_hbm, out_hbm,
                 cidx_v, ra, rb, buf_a, buf_b, acc_sh,
                 gsa, gsb, ssa, ssb, ila, ilb):
    cid = lax.axis_index("c")
    sid = lax.axis_index("s")
    wid = cid * NS + sid

    def fill_zero(i, _):
        for k in range(OUT // 16):
            buf_a[i, pl.ds(k * 16, 16)] = jnp.zeros((16,), jnp.float32)
        return 0

    lax.fori_loop(0, K, fill_zero, 0)
    for c in range(NPT // K):
        pltpu.sync_copy(buf_a, acc_sh.at[pl.ds(sid * NPT + c * K, K)])
    rem = NPT - (NPT // K) * K
    pltpu.sync_copy(buf_a.at[pl.ds(0, rem)],
                    acc_sh.at[pl.ds(sid * NPT + (NPT // K) * K, rem)])
    pltpu.sync_copy(col_hbm.at[pl.ds(wid * CHP, CHP)], cidx_v)
    plsc.subcore_barrier()

    def _rslice(j):
        start = pl.multiple_of(wid * EPW + j * K, 8)
        return row_hbm.at[pl.ds(start, K)]

    def ridx_load(j, r, sem):
        return pltpu.async_copy(_rslice(j), r, sem)

    def ridx_wait(j, r, sem):
        pltpu.make_async_copy(_rslice(j), r, sem).wait()

    def gather(r, buf, sem):
        return pltpu.async_copy(ys_hbm.at[r], buf, sem)

    def gather_wait(r, buf, sem):
        pltpu.make_async_copy(ys_hbm.at[r], buf, sem).wait()

    def scatter(j, buf, sem):
        return pltpu.async_copy(buf, acc_sh.at[cidx_v.at[j]], sem, add=True)

    def scatter_wait(j, buf, sem):
        pltpu.make_async_copy(buf, acc_sh.at[cidx_v.at[j]], sem).wait()

    pltpu.sync_copy(_rslice(0), ra)
    gather(ra, buf_a, gsa)
    ridx_load(1, rb, ilb)
    gather_wait(ra, buf_a, gsa)
    ridx_wait(1, rb, ilb)
    gather(rb, buf_b, gsb)
    ridx_load(2, ra, ila)
    scatter(0, buf_a, ssa)

    def pair(t, _):
        j = 2 * t + 1
        gather_wait(rb, buf_b, gsb)
        scatter_wait(j - 1, buf_a, ssa)
        ridx_wait(j + 1, ra, ila)
        gather(ra, buf_a, gsa)
        ridx_load(j + 2, rb, ilb)
        scatter(j, buf_b, ssb)
        gather_wait(ra, buf_a, gsa)
        scatter_wait(j, buf_b, ssb)
        ridx_wait(j + 2, rb, ilb)
        gather(rb, buf_b, gsb)
        ridx_load(j + 3, ra, ila)
        scatter(j + 1, buf_a, ssa)
        return 0

    lax.fori_loop(0, (CH - 3) // 2, pair, 0)
    gather_wait(rb, buf_b, gsb)
    scatter_wait(CH - 3, buf_a, ssa)
    ridx_wait(CH - 1, ra, ila)
    gather(ra, buf_a, gsa)
    scatter(CH - 2, buf_b, ssb)
    gather_wait(ra, buf_a, gsa)
    scatter_wait(CH - 2, buf_b, ssb)
    scatter(CH - 1, buf_a, ssa)
    scatter_wait(CH - 1, buf_a, ssa)

    plsc.subcore_barrier()
    pltpu.sync_copy(acc_sh.at[pl.ds(sid * NPT, NPT)], out_hbm.at[wid])


BLK = 1000


def _mm1_body(x_ref, mem_ref, wt1_ref, wt2_ref, whht_ref, bhh_ref,
              xw_ref, gh_ref):
    xw = jnp.dot(x_ref[...], wt1_ref[...],
                 preferred_element_type=jnp.float32, precision=HIGHEST)
    xw = xw + jnp.dot(mem_ref[...], wt2_ref[...],
                      preferred_element_type=jnp.float32, precision=HIGHEST)
    xw_ref[...] = xw
    gh_ref[...] = jnp.dot(mem_ref[...], whht_ref[...],
                          preferred_element_type=jnp.float32,
                          precision=HIGHEST) + bhh_ref[...]


def _tc_mm1(x, memory, wt1, wt2, whht, bhh):
    grid = (N // BLK,)
    return pl.pallas_call(
        _mm1_body,
        grid=grid,
        in_specs=[
            pl.BlockSpec((BLK, IN_CH), lambda i: (i, 0)),
            pl.BlockSpec((BLK, MEM), lambda i: (i, 0)),
            pl.BlockSpec((IN_CH, OUT), lambda i: (0, 0)),
            pl.BlockSpec((MEM, OUT), lambda i: (0, 0)),
            pl.BlockSpec((MEM, 3 * MEM), lambda i: (0, 0)),
            pl.BlockSpec((1, 3 * MEM), lambda i: (0, 0)),
        ],
        out_specs=[
            pl.BlockSpec((BLK, OUT), lambda i: (i, 0)),
            pl.BlockSpec((BLK, 3 * MEM), lambda i: (i, 0)),
        ],
        out_shape=[
            jax.ShapeDtypeStruct((N, OUT), jnp.float32),
            jax.ShapeDtypeStruct((N, 3 * MEM), jnp.float32),
        ],
    )(x, memory, wt1, wt2, whht, bhh)


def _scale_body(d0_ref, d1_ref, xw_ref, ys_ref, dis_ref):
    deg = d0_ref[:, 0:1] + d1_ref[:, 0:1] + 1.0
    dis = jax.lax.rsqrt(deg)
    ys_ref[...] = dis * xw_ref[...]
    dis_ref[...] = dis


def _tc_scale(deg0, deg1, xw):
    grid = (N // BLK,)
    return pl.pallas_call(
        _scale_body,
        grid=grid,
        in_specs=[
            pl.BlockSpec((BLK, OUT), lambda i: (i, 0)),
            pl.BlockSpec((BLK, OUT), lambda i: (i, 0)),
            pl.BlockSpec((BLK, OUT), lambda i: (i, 0)),
        ],
        out_specs=[
            pl.BlockSpec((BLK, OUT), lambda i: (i, 0)),
            pl.BlockSpec((BLK, 1), lambda i: (i, 0)),
        ],
        out_shape=[
            jax.ShapeDtypeStruct((N, OUT), jnp.float32),
            jax.ShapeDtypeStruct((N, 1), jnp.float32),
        ],
    )(deg0, deg1, xw)


def _final_body(a0_ref, a1_ref, ys_ref, dis_ref, mem_ref, gh_ref,
                wiht_ref, bih_ref, gb_ref, nm_ref, msg_ref):
    msg = dis_ref[...] * (a0_ref[...] + a1_ref[...] + ys_ref[...]) + gb_ref[...]
    msg_ref[...] = msg
    gi = jnp.dot(msg, wiht_ref[...],
                 preferred_element_type=jnp.float32,
                 precision=HIGHEST) + bih_ref[...]
    gh = gh_ref[...]
    r = jax.nn.sigmoid(gi[:, 0:MEM] + gh[:, 0:MEM])
    z = jax.nn.sigmoid(gi[:, MEM:2 * MEM] + gh[:, MEM:2 * MEM])
    ng = jnp.tanh(gi[:, 2 * MEM:] + r * gh[:, 2 * MEM:])
    nm_ref[...] = (1.0 - z) * ng + z * mem_ref[...]


def _tc_final(acc0, acc1, ys, dis, memory, gh, wiht, bih, gbias):
    grid = (N // BLK,)
    return pl.pallas_call(
        _final_body,
        grid=grid,
        in_specs=[
            pl.BlockSpec((BLK, OUT), lambda i: (i, 0)),
            pl.BlockSpec((BLK, OUT), lambda i: (i, 0)),
            pl.BlockSpec((BLK, OUT), lambda i: (i, 0)),
            pl.BlockSpec((BLK, 1), lambda i: (i, 0)),
            pl.BlockSpec((BLK, MEM), lambda i: (i, 0)),
            pl.BlockSpec((BLK, 3 * MEM), lambda i: (i, 0)),
            pl.BlockSpec((OUT, 3 * MEM), lambda i: (0, 0)),
            pl.BlockSpec((1, 3 * MEM), lambda i: (0, 0)),
            pl.BlockSpec((1, OUT), lambda i: (0, 0)),
        ],
        out_specs=[
            pl.BlockSpec((BLK, MEM), lambda i: (i, 0)),
            pl.BlockSpec((BLK, OUT), lambda i: (i, 0)),
        ],
        out_shape=[
            jax.ShapeDtypeStruct((N, MEM), jnp.float32),
            jax.ShapeDtypeStruct((N, OUT), jnp.float32),
        ],
    )(acc0, acc1, ys, dis, memory, gh, wiht, bih, gbias)


def kernel(x, edge_index, memory, gcn_weight, gcn_bias, w_ih, w_hh, b_ih, b_hh):
    row = edge_index[0].astype(jnp.int32)
    col = edge_index[1].astype(jnp.int32).reshape(NW, CH, K)
    col = jnp.pad(col, ((0, 0), (0, CHP - CH), (0, 0))).reshape(NW * CHP, K)

    hist = _sc_histogram(col)
    deg0 = hist[:NS].reshape(N, OUT)
    deg1 = hist[NS:].reshape(N, OUT)

    wt1 = gcn_weight[:, :IN_CH].T
    wt2 = gcn_weight[:, IN_CH:].T
    whht = w_hh.T
    wiht = w_ih.T
    xw, gh = _tc_mm1(x, memory, wt1, wt2, whht, b_hh.reshape(1, -1))

    ys, dis = _tc_scale(deg0, deg1, xw)

    acc = _sc_edge_sum(ys, row, col)
    acc0 = acc[:NS].reshape(N, OUT)
    acc1 = acc[NS:].reshape(N, OUT)

    new_memory, message = _tc_final(
        acc0, acc1, ys, dis, memory, gh, wiht,
        b_ih.reshape(1, -1), gcn_bias.reshape(1, -1))
    return (new_memory, message)


# speedup vs baseline: 22.8838x; 1.0404x over previous
import functools

import jax
import jax.numpy as jnp
from jax import lax
from jax.experimental import pallas as pl
from jax.experimental.pallas import tpu as pltpu
from jax.experimental.pallas import tpu_sc as plsc

N = 10000
E = 320000
IN_CH = 128
MEM = 128
OUT = 128

NC = 2
NS = 16
NW = NC * NS
EPW = E // NW
NPT = N // NS
ZB = 125
K2 = 128
CH2 = 79
CHP2 = 80
EPT = CH2 * K2
ACC_N = N + NS
WH = 8

_mesh = plsc.VectorSubcoreMesh(core_axis_name="c", subcore_axis_name="s")

HIGHEST = jax.lax.Precision.HIGHEST


@functools.partial(
    pl.kernel,
    out_type=jax.ShapeDtypeStruct((NW, NPT, OUT), jnp.float32),
    mesh=_mesh,
    scratch_types=[
        pltpu.VMEM((CHP2, K2), jnp.int32),
        pltpu.VMEM((K2, OUT), jnp.float32),
        pltpu.VMEM((ZB, OUT), jnp.float32),
        pltpu.VMEM_SHARED((ACC_N, OUT), jnp.float32),
        pltpu.SemaphoreType.DMA,
    ],
)
def _sc_histogram(col_hbm, out_hbm, cidx_v, ones_v, zero_v, acc_sh, sem):
    cid = lax.axis_index("c")
    sid = lax.axis_index("s")
    wid = cid * NS + sid

    def fill_ones(i, _):
        for k in range(OUT // 16):
            ones_v[i, pl.ds(k * 16, 16)] = jnp.ones((16,), jnp.float32)
        return 0

    lax.fori_loop(0, K2, fill_ones, 0)

    def fill_zero(i, _):
        for k in range(OUT // 16):
            zero_v[i, pl.ds(k * 16, 16)] = jnp.zeros((16,), jnp.float32)
        return 0

    lax.fori_loop(0, ZB, fill_zero, 0)
    for c in range(NPT // ZB):
        pltpu.sync_copy(zero_v, acc_sh.at[pl.ds(sid * NPT + c * ZB, ZB)])
    pltpu.sync_copy(col_hbm.at[pl.ds(wid * CHP2, CHP2)], cidx_v)
    plsc.subcore_barrier()

    for w in range(WH):
        pltpu.async_copy(ones_v, acc_sh.at[cidx_v.at[w]], sem, add=True)

    def chunk(j, _):
        pltpu.make_async_copy(ones_v, acc_sh.at[cidx_v.at[j - WH]], sem).wait()
        pltpu.async_copy(ones_v, acc_sh.at[cidx_v.at[j]], sem, add=True)
        return 0

    lax.fori_loop(WH, CH2, chunk, 0)
    for w in range(WH):
        pltpu.make_async_copy(
            ones_v, acc_sh.at[cidx_v.at[CH2 - WH + w]], sem).wait()
    plsc.subcore_barrier()

    pltpu.sync_copy(acc_sh.at[pl.ds(sid * NPT, NPT)], out_hbm.at[wid])


@functools.partial(
    pl.kernel,
    out_type=jax.ShapeDtypeStruct((NW, NPT, OUT), jnp.float32),
    mesh=_mesh,
    scratch_types=[
        pltpu.VMEM((CHP2, K2), jnp.int32),
        pltpu.VMEM((K2,), jnp.int32),
        pltpu.VMEM((K2,), jnp.int32),
        pltpu.VMEM((K2, OUT), jnp.float32),
        pltpu.VMEM((K2, OUT), jnp.float32),
        pltpu.VMEM_SHARED((ACC_N, OUT), jnp.float32),
        pltpu.SemaphoreType.DMA,
        pltpu.SemaphoreType.DMA,
        pltpu.SemaphoreType.DMA,
        pltpu.SemaphoreType.DMA,
        pltpu.SemaphoreType.DMA,
        pltpu.SemaphoreType.DMA,
    ],
)
def _sc_edge_sum(ys_hbm, row_hbm, col_hbm, out_hbm,
                 cidx_v, ra, rb, buf_a, buf_b, acc_sh,
                 gsa, gsb, ssa, ssb, ila, ilb):
    cid = lax.axis_index("c")
    sid = lax.axis_index("s")
    wid = cid * NS + sid

    def fill_zero(i, _):
        for k in range(OUT // 16):
            buf_a[i, pl.ds(k * 16, 16)] = jnp.zeros((16,), jnp.float32)
        return 0

    lax.fori_loop(0, K2, fill_zero, 0)
    for c in range(NPT // K2):
        pltpu.sync_copy(buf_a, acc_sh.at[pl.ds(sid * NPT + c * K2, K2)])
    rem = NPT - (NPT // K2) * K2
    pltpu.sync_copy(buf_a.at[pl.ds(0, rem)],
                    acc_sh.at[pl.ds(sid * NPT + (NPT // K2) * K2, rem)])
    pltpu.sync_copy(col_hbm.at[pl.ds(wid * CHP2, CHP2)], cidx_v)
    plsc.subcore_barrier()

    def _rslice(j):
        start = pl.multiple_of(wid * EPT + j * K2, 8)
        return row_hbm.at[pl.ds(start, K2)]

    def ridx_load(j, r, sem):
        return pltpu.async_copy(_rslice(j), r, sem)

    def ridx_wait(j, r, sem):
        pltpu.make_async_copy(_rslice(j), r, sem).wait()

    def gather(r, buf, sem):
        return pltpu.async_copy(ys_hbm.at[r], buf, sem)

    def gather_wait(r, buf, sem):
        pltpu.make_async_copy(ys_hbm.at[r], buf, sem).wait()

    def scatter(j, buf, sem):
        return pltpu.async_copy(buf, acc_sh.at[cidx_v.at[j]], sem, add=True)

    def scatter_wait(j, buf, sem):
        pltpu.make_async_copy(buf, acc_sh.at[cidx_v.at[j]], sem).wait()

    pltpu.sync_copy(_rslice(0), ra)
    gather(ra, buf_a, gsa)
    ridx_load(1, rb, ilb)
    gather_wait(ra, buf_a, gsa)
    ridx_wait(1, rb, ilb)
    gather(rb, buf_b, gsb)
    ridx_load(2, ra, ila)
    scatter(0, buf_a, ssa)

    def pair(t, _):
        j = 2 * t + 1
        gather_wait(rb, buf_b, gsb)
        scatter_wait(j - 1, buf_a, ssa)
        ridx_wait(j + 1, ra, ila)
        gather(ra, buf_a, gsa)
        ridx_load(j + 2, rb, ilb)
        scatter(j, buf_b, ssb)
        gather_wait(ra, buf_a, gsa)
        scatter_wait(j, buf_b, ssb)
        ridx_wait(j + 2, rb, ilb)
        gather(rb, buf_b, gsb)
        ridx_load(j + 3, ra, ila)
        scatter(j + 1, buf_a, ssa)
        return 0

    lax.fori_loop(0, (CH2 - 3) // 2, pair, 0)
    gather_wait(rb, buf_b, gsb)
    scatter_wait(CH2 - 3, buf_a, ssa)
    ridx_wait(CH2 - 1, ra, ila)
    gather(ra, buf_a, gsa)
    scatter(CH2 - 2, buf_b, ssb)
    gather_wait(ra, buf_a, gsa)
    scatter_wait(CH2 - 2, buf_b, ssb)
    scatter(CH2 - 1, buf_a, ssa)
    scatter_wait(CH2 - 1, buf_a, ssa)

    plsc.subcore_barrier()
    pltpu.sync_copy(acc_sh.at[pl.ds(sid * NPT, NPT)], out_hbm.at[wid])


BLK = 1000


def _mm1_body(x_ref, mem_ref, wt1_ref, wt2_ref, xw_ref):
    xw = jnp.dot(x_ref[...], wt1_ref[...],
                 preferred_element_type=jnp.float32, precision=HIGHEST)
    xw = xw + jnp.dot(mem_ref[...], wt2_ref[...],
                      preferred_element_type=jnp.float32, precision=HIGHEST)
    xw_ref[...] = xw


def _tc_mm1(x, memory, wt1, wt2):
    grid = (N // BLK,)
    return pl.pallas_call(
        _mm1_body,
        grid=grid,
        in_specs=[
            pl.BlockSpec((BLK, IN_CH), lambda i: (i, 0)),
            pl.BlockSpec((BLK, MEM), lambda i: (i, 0)),
            pl.BlockSpec((IN_CH, OUT), lambda i: (0, 0)),
            pl.BlockSpec((MEM, OUT), lambda i: (0, 0)),
        ],
        out_specs=pl.BlockSpec((BLK, OUT), lambda i: (i, 0)),
        out_shape=jax.ShapeDtypeStruct((N, OUT), jnp.float32),
    )(x, memory, wt1, wt2)


def _scale_body(d0_ref, d1_ref, xw_ref, ys_ref, dis_ref):
    deg = d0_ref[:, 0:1] + d1_ref[:, 0:1] + 1.0
    dis = jax.lax.rsqrt(deg)
    ys_ref[...] = dis * xw_ref[...]
    dis_ref[...] = dis


def _tc_scale(deg0, deg1, xw):
    grid = (N // BLK,)
    return pl.pallas_call(
        _scale_body,
        grid=grid,
        in_specs=[
            pl.BlockSpec((BLK, OUT), lambda i: (i, 0)),
            pl.BlockSpec((BLK, OUT), lambda i: (i, 0)),
            pl.BlockSpec((BLK, OUT), lambda i: (i, 0)),
        ],
        out_specs=[
            pl.BlockSpec((BLK, OUT), lambda i: (i, 0)),
            pl.BlockSpec((BLK, 1), lambda i: (i, 0)),
        ],
        out_shape=[
            jax.ShapeDtypeStruct((N, OUT), jnp.float32),
            jax.ShapeDtypeStruct((N, 1), jnp.float32),
        ],
    )(deg0, deg1, xw)


def _final_body(a0_ref, a1_ref, ys_ref, dis_ref, mem_ref,
                wiht_ref, whht_ref, bih_ref, bhh_ref, gb_ref,
                nm_ref, msg_ref):
    msg = dis_ref[...] * (a0_ref[...] + a1_ref[...] + ys_ref[...]) + gb_ref[...]
    msg_ref[...] = msg
    gi = jnp.dot(msg, wiht_ref[...],
                 preferred_element_type=jnp.float32,
                 precision=HIGHEST) + bih_ref[...]
    gh = jnp.dot(mem_ref[...], whht_ref[...],
                 preferred_element_type=jnp.float32,
                 precision=HIGHEST) + bhh_ref[...]
    r = jax.nn.sigmoid(gi[:, 0:MEM] + gh[:, 0:MEM])
    z = jax.nn.sigmoid(gi[:, MEM:2 * MEM] + gh[:, MEM:2 * MEM])
    ng = jnp.tanh(gi[:, 2 * MEM:] + r * gh[:, 2 * MEM:])
    nm_ref[...] = (1.0 - z) * ng + z * mem_ref[...]


def _tc_final(acc0, acc1, ys, dis, memory, wiht, whht, bih, bhh, gbias):
    grid = (N // BLK,)
    return pl.pallas_call(
        _final_body,
        grid=grid,
        in_specs=[
            pl.BlockSpec((BLK, OUT), lambda i: (i, 0)),
            pl.BlockSpec((BLK, OUT), lambda i: (i, 0)),
            pl.BlockSpec((BLK, OUT), lambda i: (i, 0)),
            pl.BlockSpec((BLK, 1), lambda i: (i, 0)),
            pl.BlockSpec((BLK, MEM), lambda i: (i, 0)),
            pl.BlockSpec((OUT, 3 * MEM), lambda i: (0, 0)),
            pl.BlockSpec((MEM, 3 * MEM), lambda i: (0, 0)),
            pl.BlockSpec((1, 3 * MEM), lambda i: (0, 0)),
            pl.BlockSpec((1, 3 * MEM), lambda i: (0, 0)),
            pl.BlockSpec((1, OUT), lambda i: (0, 0)),
        ],
        out_specs=[
            pl.BlockSpec((BLK, MEM), lambda i: (i, 0)),
            pl.BlockSpec((BLK, OUT), lambda i: (i, 0)),
        ],
        out_shape=[
            jax.ShapeDtypeStruct((N, MEM), jnp.float32),
            jax.ShapeDtypeStruct((N, OUT), jnp.float32),
        ],
    )(acc0, acc1, ys, dis, memory, wiht, whht, bih, bhh, gbias)


def kernel(x, edge_index, memory, gcn_weight, gcn_bias, w_ih, w_hh, b_ih, b_hh):
    row = edge_index[0].astype(jnp.int32)
    col = edge_index[1].astype(jnp.int32)

    pad = EPT - EPW
    wids = jnp.arange(NW, dtype=jnp.int32)[:, None]
    prange = jnp.arange(pad, dtype=jnp.int32)[None, :]
    row_pad = (wids * 97 + prange * 131) % N
    col_pad = N + (wids % NS) + jnp.zeros_like(prange)
    row_p = jnp.concatenate(
        [row.reshape(NW, EPW), row_pad], axis=1).reshape(NW * EPT)
    col_p = jnp.concatenate(
        [col.reshape(NW, EPW), col_pad], axis=1).reshape(NW, CH2, K2)
    col_p = jnp.pad(col_p, ((0, 0), (0, CHP2 - CH2), (0, 0)))
    col_p = col_p.reshape(NW * CHP2, K2)

    hist = _sc_histogram(col_p)
    deg0 = hist[:NS].reshape(N, OUT)
    deg1 = hist[NS:].reshape(N, OUT)

    wt1 = gcn_weight[:, :IN_CH].T
    wt2 = gcn_weight[:, IN_CH:].T
    whht = w_hh.T
    wiht = w_ih.T
    xw = _tc_mm1(x, memory, wt1, wt2)

    ys, dis = _tc_scale(deg0, deg1, xw)

    acc = _sc_edge_sum(ys, row_p, col_p)
    acc0 = acc[:NS].reshape(N, OUT)
    acc1 = acc[NS:].reshape(N, OUT)

    new_memory, message = _tc_final(
        acc0, acc1, ys, dis, memory, wiht, whht,
        b_ih.reshape(1, -1), b_hh.reshape(1, -1), gcn_bias.reshape(1, -1))
    return (new_memory, message)
